# baseline (speedup 1.0000x reference)
import functools

import jax
import jax.numpy as jnp
from jax import lax
from jax.experimental import pallas as pl
from jax.experimental.pallas import tpu as pltpu
from jax.experimental.pallas import tpu_sc as plsc

_S = 30.0
_NC, _NS, _L = 2, 16, 16
_NW = _NC * _NS


def _make_sc_dense(b, c):
    rpw = b // _NW
    ng = rpw // _L
    mesh = plsc.VectorSubcoreMesh(core_axis_name="c", subcore_axis_name="s")

    @functools.partial(
        pl.kernel,
        mesh=mesh,
        out_type=[
            jax.ShapeDtypeStruct((b,), jnp.float32),
            jax.ShapeDtypeStruct((b,), jnp.float32),
        ],
        scratch_types=[
            pltpu.VMEM((rpw * c,), jnp.float32),
            pltpu.VMEM((rpw,), jnp.int32),
            pltpu.VMEM((128,), jnp.float32),
            pltpu.VMEM((rpw,), jnp.float32),
            pltpu.VMEM((rpw,), jnp.float32),
        ],
        compiler_params=pltpu.CompilerParams(needs_layout_passes=False),
    )
    def sc_dense(x_hbm, ml_hbm, t_hbm, z_out, w_out, buf, t_v, ml_v, zv, wv):
        wid = lax.axis_index("s") * _NC + lax.axis_index("c")
        row0 = wid * rpw
        pltpu.sync_copy(x_hbm.at[pl.ds(row0 * c, rpw * c)], buf)
        pltpu.sync_copy(t_hbm.at[pl.ds(row0, rpw)], t_v)
        pltpu.sync_copy(ml_hbm, ml_v)
        iota = lax.broadcasted_iota(jnp.int32, (_L,), 0)

        def group(g, carry):
            base = g * (_L * c) + iota * c
            v = plsc.load_gather(buf, [base])
            m16 = v
            for cc in range(1, c):
                v = plsc.load_gather(buf, [base + cc])
                m16 = jnp.maximum(m16, v)
            e16 = jnp.zeros((_L,), jnp.float32)
            for cc in range(c):
                v = plsc.load_gather(buf, [base + cc])
                e16 = e16 + jnp.exp((v - m16) * _S)
            t16 = t_v[pl.ds(g * _L, _L)]
            p16 = plsc.load_gather(buf, [base + t16])
            bm16 = plsc.load_gather(ml_v, [t16])
            q16 = p16 - bm16
            z16 = e16 - jnp.exp((p16 - m16) * _S) + jnp.exp((q16 - m16) * _S)
            w16 = _S * m16 - _S * q16
            zv[pl.ds(g * _L, _L)] = z16
            wv[pl.ds(g * _L, _L)] = w16
            return carry

        lax.fori_loop(0, ng, group, 0)
        pltpu.sync_copy(zv, z_out.at[pl.ds(row0, rpw)])
        pltpu.sync_copy(wv, w_out.at[pl.ds(row0, rpw)])

    return sc_dense


def _combine_body(z_ref, w_ref, out_ref):
    n = z_ref.shape[0] * z_ref.shape[1]
    lossb = jnp.log(z_ref[...]) + w_ref[...]
    out_ref[...] = (jnp.sum(lossb) * (1.0 / n))[None, None]


def kernel(x, m_list, target):
    b, c = x.shape
    xflat = x.reshape(-1)
    t32 = target.astype(jnp.int32)
    m128 = jnp.pad(m_list, (0, 128 - c))
    z, w = _make_sc_dense(b, c)(xflat, m128, t32)
    r = 8
    q = b // r
    out = pl.pallas_call(
        _combine_body,
        out_shape=jax.ShapeDtypeStruct((1, 1), jnp.float32),
    )(z.reshape(r, q), w.reshape(r, q))
    return out[0, 0]


# speedup vs baseline: 1.1082x; 1.1082x over previous
import functools

import jax
import jax.numpy as jnp
from jax import lax
from jax.experimental import pallas as pl
from jax.experimental.pallas import tpu as pltpu
from jax.experimental.pallas import tpu_sc as plsc

_S = 30.0
_NC, _NS, _L = 2, 16, 16
_NW = _NC * _NS
_B_SC = 4096


def _make_sc_dense(b, c, b_sc):
    rpw = b_sc // _NW
    ng = rpw // _L
    row_base = b - b_sc
    mesh = plsc.VectorSubcoreMesh(core_axis_name="c", subcore_axis_name="s")

    @functools.partial(
        pl.kernel,
        mesh=mesh,
        out_type=[
            jax.ShapeDtypeStruct((b_sc,), jnp.float32),
            jax.ShapeDtypeStruct((b_sc,), jnp.float32),
        ],
        scratch_types=[
            pltpu.VMEM((rpw * c,), jnp.float32),
            pltpu.VMEM((rpw,), jnp.int32),
            pltpu.VMEM((128,), jnp.float32),
            pltpu.VMEM((rpw,), jnp.float32),
            pltpu.VMEM((rpw,), jnp.float32),
        ],
        compiler_params=pltpu.CompilerParams(needs_layout_passes=False),
    )
    def sc_dense(x_hbm, ml_hbm, t_hbm, z_out, w_out, buf, t_v, ml_v, zv, wv):
        wid = lax.axis_index("s") * _NC + lax.axis_index("c")
        out0 = wid * rpw
        row0 = row_base + out0
        pltpu.sync_copy(x_hbm.at[pl.ds(row0 * c, rpw * c)], buf)
        pltpu.sync_copy(t_hbm.at[pl.ds(row0, rpw)], t_v)
        pltpu.sync_copy(ml_hbm, ml_v)
        iota = lax.broadcasted_iota(jnp.int32, (_L,), 0)

        def group(g, carry):
            base = g * (_L * c) + iota * c
            v = plsc.load_gather(buf, [base])
            m16 = v
            for cc in range(1, c):
                v = plsc.load_gather(buf, [base + cc])
                m16 = jnp.maximum(m16, v)
            e16 = jnp.zeros((_L,), jnp.float32)
            for cc in range(c):
                v = plsc.load_gather(buf, [base + cc])
                e16 = e16 + jnp.exp((v - m16) * _S)
            t16 = t_v[pl.ds(g * _L, _L)]
            p16 = plsc.load_gather(buf, [base + t16])
            bm16 = plsc.load_gather(ml_v, [t16])
            q16 = p16 - bm16
            z16 = e16 - jnp.exp((p16 - m16) * _S) + jnp.exp((q16 - m16) * _S)
            w16 = _S * m16 - _S * q16
            zv[pl.ds(g * _L, _L)] = z16
            wv[pl.ds(g * _L, _L)] = w16
            return carry

        lax.fori_loop(0, ng, group, 0)
        pltpu.sync_copy(zv, z_out.at[pl.ds(out0, rpw)])
        pltpu.sync_copy(wv, w_out.at[pl.ds(out0, rpw)])

    return sc_dense


def _tc_body(x_ref, m_ref, t_ref, out_ref):
    i = pl.program_id(0)
    br, c = x_ref.shape
    x = x_ref[...]
    t = t_ref[0, 0, :]
    mrow = m_ref[0, :]

    ones = jnp.ones((c, 1), jnp.float32)

    def msum(v):
        return jnp.dot(v, ones, preferred_element_type=jnp.float32)[:, 0]

    col = lax.broadcasted_iota(jnp.int32, (br, c), 1)
    tmask = col == t[:, None]
    p = msum(jnp.where(tmask, x, 0.0))
    bm = msum(jnp.where(tmask, mrow[None, :], 0.0))

    rmax = jnp.max(x, axis=1)
    expd = jnp.exp(_S * x - (_S * rmax)[:, None])
    e = msum(expd)
    t1 = msum(jnp.where(tmask, expd, 0.0))
    z = e - t1 + jnp.exp(_S * (p - bm - rmax))
    lossb = _S * rmax + jnp.log(z) - _S * (p - bm)

    part = jnp.sum(lossb)[None, None]

    @pl.when(i == 0)
    def _init():
        out_ref[...] = jnp.zeros((1, 1), jnp.float32)

    out_ref[...] += part


def _combine_body(z_ref, w_ref, part_ref, out_ref, *, b):
    lossb = jnp.log(z_ref[...]) + w_ref[...]
    out_ref[...] = ((jnp.sum(lossb) + part_ref[0, 0]) * (1.0 / b))[None, None]


def kernel(x, m_list, target):
    b, c = x.shape
    b_tc = b - _B_SC
    xflat = x.reshape(-1)
    t32 = target.astype(jnp.int32)
    m128 = jnp.pad(m_list, (0, 128 - c))
    m2 = m_list.reshape(1, c)

    z, w = _make_sc_dense(b, c, _B_SC)(xflat, m128, t32)

    br = 4096
    nb = b_tc // br
    t3 = t32[:b_tc].reshape(nb, 1, br)
    part = pl.pallas_call(
        _tc_body,
        grid=(nb,),
        in_specs=[
            pl.BlockSpec((br, c), lambda i: (i, 0)),
            pl.BlockSpec((1, c), lambda i: (0, 0)),
            pl.BlockSpec((1, 1, br), lambda i: (i, 0, 0)),
        ],
        out_specs=pl.BlockSpec((1, 1), lambda i: (0, 0)),
        out_shape=jax.ShapeDtypeStruct((1, 1), jnp.float32),
    )(x, m2, t3)

    r = 8
    q = _B_SC // r
    out = pl.pallas_call(
        functools.partial(_combine_body, b=b),
        out_shape=jax.ShapeDtypeStruct((1, 1), jnp.float32),
    )(z.reshape(r, q), w.reshape(r, q), part)
    return out[0, 0]


# speedup vs baseline: 1.4794x; 1.3349x over previous
import functools

import jax
import jax.numpy as jnp
from jax import lax
from jax.experimental import pallas as pl
from jax.experimental.pallas import tpu as pltpu
from jax.experimental.pallas import tpu_sc as plsc

_S = 30.0
_NC, _NS, _L = 2, 16, 16
_NW = _NC * _NS
_B_SC = 4096


def _make_sc_dense(b, c, b_sc):
    rpw = b_sc // _NW
    ng = rpw // _L
    row_base = b - b_sc
    mesh = plsc.VectorSubcoreMesh(core_axis_name="c", subcore_axis_name="s")

    @functools.partial(
        pl.kernel,
        mesh=mesh,
        out_type=[
            jax.ShapeDtypeStruct((b_sc,), jnp.float32),
            jax.ShapeDtypeStruct((b_sc,), jnp.float32),
        ],
        scratch_types=[
            pltpu.VMEM((rpw, c), jnp.float32),
            pltpu.VMEM((rpw,), jnp.int32),
            pltpu.VMEM((128,), jnp.float32),
            pltpu.VMEM((rpw,), jnp.float32),
            pltpu.VMEM((rpw,), jnp.float32),
        ],
        compiler_params=pltpu.CompilerParams(
            needs_layout_passes=False, use_tc_tiling_on_sc=True),
    )
    def sc_dense(x_hbm, ml_hbm, t_hbm, z_out, w_out, buf, t_v, ml_v, zv, wv):
        wid = lax.axis_index("s") * _NC + lax.axis_index("c")
        out0 = wid * rpw
        row0 = row_base + out0
        pltpu.sync_copy(x_hbm.at[pl.ds(row0, rpw)], buf)
        pltpu.sync_copy(t_hbm.at[pl.ds(row0, rpw)], t_v)
        pltpu.sync_copy(ml_hbm, ml_v)
        iota = lax.broadcasted_iota(jnp.int32, (_L,), 0)

        def group(g, carry):
            rowids = g * _L + iota
            v = plsc.load_gather(buf, [rowids, jnp.zeros((_L,), jnp.int32)])
            m16 = v
            for cc in range(1, c):
                v = plsc.load_gather(buf, [rowids, jnp.full((_L,), cc, jnp.int32)])
                m16 = jnp.maximum(m16, v)
            e16 = jnp.zeros((_L,), jnp.float32)
            for cc in range(c):
                v = plsc.load_gather(buf, [rowids, jnp.full((_L,), cc, jnp.int32)])
                e16 = e16 + jnp.exp((v - m16) * _S)
            t16 = t_v[pl.ds(g * _L, _L)]
            p16 = plsc.load_gather(buf, [rowids, t16])
            bm16 = plsc.load_gather(ml_v, [t16])
            q16 = p16 - bm16
            z16 = e16 - jnp.exp((p16 - m16) * _S) + jnp.exp((q16 - m16) * _S)
            w16 = _S * m16 - _S * q16
            zv[pl.ds(g * _L, _L)] = z16
            wv[pl.ds(g * _L, _L)] = w16
            return carry

        lax.fori_loop(0, ng, group, 0)
        pltpu.sync_copy(zv, z_out.at[pl.ds(out0, rpw)])
        pltpu.sync_copy(wv, w_out.at[pl.ds(out0, rpw)])

    return sc_dense


def _tc_body(x_ref, m_ref, t_ref, out_ref):
    i = pl.program_id(0)
    br, c = x_ref.shape
    x = x_ref[...]
    t = t_ref[0, 0, :]
    mrow = m_ref[0, :]

    ones = jnp.ones((c, 1), jnp.float32)

    def msum(v):
        return jnp.dot(v, ones, preferred_element_type=jnp.float32)[:, 0]

    col = lax.broadcasted_iota(jnp.int32, (br, c), 1)
    tmask = col == t[:, None]
    p = msum(jnp.where(tmask, x, 0.0))
    bm = msum(jnp.where(tmask, mrow[None, :], 0.0))

    rmax = jnp.max(x, axis=1)
    expd = jnp.exp(_S * x - (_S * rmax)[:, None])
    e = msum(expd)
    t1 = msum(jnp.where(tmask, expd, 0.0))
    z = e - t1 + jnp.exp(_S * (p - bm - rmax))
    lossb = _S * rmax + jnp.log(z) - _S * (p - bm)

    part = jnp.sum(lossb)[None, None]

    @pl.when(i == 0)
    def _init():
        out_ref[...] = jnp.zeros((1, 1), jnp.float32)

    out_ref[...] += part


def _combine_body(z_ref, w_ref, part_ref, out_ref, *, b):
    lossb = jnp.log(z_ref[...]) + w_ref[...]
    out_ref[...] = ((jnp.sum(lossb) + part_ref[0, 0]) * (1.0 / b))[None, None]


def kernel(x, m_list, target):
    b, c = x.shape
    b_tc = b - _B_SC
    t32 = target.astype(jnp.int32)
    m128 = jnp.pad(m_list, (0, 128 - c))
    m2 = m_list.reshape(1, c)

    z, w = _make_sc_dense(b, c, _B_SC)(x, m128, t32)

    br = 4096
    nb = b_tc // br
    t3 = t32[:b_tc].reshape(nb, 1, br)
    part = pl.pallas_call(
        _tc_body,
        grid=(nb,),
        in_specs=[
            pl.BlockSpec((br, c), lambda i: (i, 0)),
            pl.BlockSpec((1, c), lambda i: (0, 0)),
            pl.BlockSpec((1, 1, br), lambda i: (i, 0, 0)),
        ],
        out_specs=pl.BlockSpec((1, 1), lambda i: (0, 0)),
        out_shape=jax.ShapeDtypeStruct((1, 1), jnp.float32),
    )(x, m2, t3)

    r = 8
    q = _B_SC // r
    out = pl.pallas_call(
        functools.partial(_combine_body, b=b),
        out_shape=jax.ShapeDtypeStruct((1, 1), jnp.float32),
    )(z.reshape(r, q), w.reshape(r, q), part)
    return out[0, 0]


# speedup vs baseline: 2.5663x; 1.7347x over previous
import jax
import jax.numpy as jnp
from jax import lax
from jax.experimental import pallas as pl

_S = 30.0


def _half_loss(x, t, mrow):
    br, c = x.shape
    ones = jnp.ones((c, 1), jnp.float32)

    def msum(v):
        return jnp.dot(v, ones, preferred_element_type=jnp.float32)[:, 0]

    col = lax.broadcasted_iota(jnp.int32, (br, c), 1)
    tmask = col == t[:, None]
    p = msum(jnp.where(tmask, x, 0.0))
    bm = msum(jnp.where(tmask, mrow[None, :], 0.0))

    rmax = jnp.max(x, axis=1)
    expd = jnp.exp(_S * x - (_S * rmax)[:, None])
    e = msum(expd)
    t1 = msum(jnp.where(tmask, expd, 0.0))
    z = e - t1 + jnp.exp(_S * (p - bm - rmax))
    lossb = _S * rmax + jnp.log(z) - _S * (p - bm)
    return jnp.sum(lossb)


def _ldam_body(xa_ref, xb_ref, m_ref, ta_ref, tb_ref, out_ref):
    i = pl.program_id(0)
    nb = pl.num_programs(0)
    br, c = xa_ref.shape
    mrow = m_ref[0, :]
    s = _half_loss(xa_ref[...], ta_ref[0, 0, :], mrow)
    s += _half_loss(xb_ref[...], tb_ref[0, 0, :], mrow)
    part = (s * (1.0 / (2 * br * nb)))[None, None]

    @pl.when(i == 0)
    def _init():
        out_ref[...] = jnp.zeros((1, 1), jnp.float32)

    out_ref[...] += part


def kernel(x, m_list, target):
    b, c = x.shape
    br = 2048
    nb = b // (2 * br)
    t32 = target.astype(jnp.int32)
    t3 = t32.reshape(2 * nb, 1, br)
    ta = t3[0::2]
    tb = t3[1::2]
    m2 = m_list.reshape(1, c)
    out = pl.pallas_call(
        _ldam_body,
        grid=(nb,),
        in_specs=[
            pl.BlockSpec((br, c), lambda i: (2 * i, 0)),
            pl.BlockSpec((br, c), lambda i: (2 * i + 1, 0)),
            pl.BlockSpec((1, c), lambda i: (0, 0)),
            pl.BlockSpec((1, 1, br), lambda i: (i, 0, 0)),
            pl.BlockSpec((1, 1, br), lambda i: (i, 0, 0)),
        ],
        out_specs=pl.BlockSpec((1, 1), lambda i: (0, 0)),
        out_shape=jax.ShapeDtypeStruct((1, 1), jnp.float32),
    )(x, x, m2, ta, tb)
    return out[0, 0]


# speedup vs baseline: 2.7994x; 1.0908x over previous
import jax
import jax.numpy as jnp
from jax import lax
from jax.experimental import pallas as pl

_S = 30.0


def _ldam_body(x_ref, m_ref, t_ref, out_ref):
    i = pl.program_id(0)
    nb = pl.num_programs(0)
    br, c = x_ref.shape
    x = x_ref[...]
    t = t_ref[0, 0, :]
    mrow = m_ref[0, :]

    ones = jnp.ones((c, 1), jnp.float32)

    def msum(v):
        return jnp.dot(v, ones, preferred_element_type=jnp.float32)[:, 0]

    col = lax.broadcasted_iota(jnp.int32, (br, c), 1)
    tmask = col == t[:, None]
    p = msum(jnp.where(tmask, x, 0.0))
    bm = msum(jnp.where(tmask, mrow[None, :], 0.0))

    rmax = jnp.max(x, axis=1)
    expd = jnp.exp(_S * x - (_S * rmax)[:, None])
    e = msum(expd)
    t1 = msum(jnp.where(tmask, expd, 0.0))
    z = e - t1 + jnp.exp(_S * (p - bm - rmax))
    lossb = _S * rmax + jnp.log(z) - _S * (p - bm)

    part = (jnp.sum(lossb) * (1.0 / (br * nb)))[None, None]

    @pl.when(i == 0)
    def _init():
        out_ref[...] = jnp.zeros((1, 1), jnp.float32)

    out_ref[...] += part


def kernel(x, m_list, target):
    b, c = x.shape
    br = 4096
    nb = b // br
    t3 = target.astype(jnp.int32).reshape(nb, 1, br)
    m2 = m_list.reshape(1, c)
    out = pl.pallas_call(
        _ldam_body,
        grid=(nb,),
        in_specs=[
            pl.BlockSpec((br, c), lambda i: (i, 0)),
            pl.BlockSpec((1, c), lambda i: (0, 0)),
            pl.BlockSpec((1, 1, br), lambda i: (i, 0, 0)),
        ],
        out_specs=pl.BlockSpec((1, 1), lambda i: (0, 0)),
        out_shape=jax.ShapeDtypeStruct((1, 1), jnp.float32),
    )(x, m2, t3)
    return out[0, 0]
